# baseline (speedup 1.0000x reference)
import functools

import jax
import jax.numpy as jnp
from jax import lax
from jax.experimental import pallas as pl
from jax.experimental.pallas import tpu as pltpu
from jax.experimental.pallas import tpu_sc as plsc

N = 10000
D = 128
E = 320000
G = 16
EPS = 1e-5

NC = 2
NS = 16
NW = NC * NS
CHUNK = 80
NCH = 125
PER_W = NCH * CHUNK
ACC_ROWS = 10112
ZROWS = ACC_ROWS // NS

R = 1000
GRID = N // R

_MESH = dict(core_axis_name="c", subcore_axis_name="s", num_cores=NC,
             num_subcores=NS)


NSLOT = 4
NITER = (NCH + NSLOT - 1) // NSLOT


def _sc_prop_body(u_hbm, src_hbm, dst_hbm, z_hbm, acc_out, acc_sh, *bufs):
    sidx = bufs[0:NSLOT]
    didx = bufs[NSLOT:2 * NSLOT]
    rows = bufs[2 * NSLOT:3 * NSLOT]
    gsem = bufs[3 * NSLOT:4 * NSLOT]
    ssem = bufs[4 * NSLOT:5 * NSLOT]
    cid = lax.axis_index("c")
    sid = lax.axis_index("s")
    pltpu.sync_copy(z_hbm, acc_sh.at[pl.ds(sid * ZROWS, ZROWS)])
    w = cid * NS + sid
    plsc.subcore_barrier()

    def idx_load(j, k):
        pltpu.sync_copy(src_hbm.at[w, j], sidx[k])
        pltpu.sync_copy(dst_hbm.at[w, j], didx[k])

    def gather_start(k):
        pltpu.async_copy(u_hbm.at[sidx[k]], rows[k], gsem[k])

    def gather_wait(k):
        pltpu.make_async_copy(u_hbm.at[sidx[k]], rows[k], gsem[k]).wait()

    def scat_start(k):
        pltpu.async_copy(rows[k], acc_sh.at[didx[k]], ssem[k], add=True)

    def scat_wait(k):
        pltpu.make_async_copy(rows[k], acc_sh.at[didx[k]], ssem[k]).wait()

    for k in range(NSLOT):
        idx_load(k, k)
        gather_start(k)

    def body(i, carry):
        for k in range(NSLOT):
            j = i * NSLOT + k

            @pl.when(j < NCH)
            def _():
                gather_wait(k)
                scat_start(k)
                jn = j + NSLOT

                @pl.when(jn < NCH)
                def _():
                    scat_wait(k)
                    idx_load(jn, k)
                    gather_start(k)
        return carry

    lax.fori_loop(0, NITER, body, 0)
    for k in range(NSLOT):
        scat_wait(k)
    plsc.subcore_barrier()
    pltpu.sync_copy(acc_sh.at[pl.ds(sid * ZROWS, ZROWS)],
                    acc_out.at[cid, pl.ds(sid * ZROWS, ZROWS)])


def _sc_deg_body(dst_hbm, ones_hbm, z_hbm, deg_out, dacc, *bufs):
    didx = bufs[0:NSLOT]
    ssem = bufs[NSLOT:2 * NSLOT]
    ones_v = bufs[2 * NSLOT]
    cid = lax.axis_index("c")
    sid = lax.axis_index("s")
    pltpu.sync_copy(z_hbm, dacc.at[pl.ds(sid * ZROWS, ZROWS)])
    w = cid * NS + sid
    pltpu.sync_copy(ones_hbm, ones_v)
    plsc.subcore_barrier()

    def scat_start(k):
        pltpu.async_copy(ones_v, dacc.at[didx[k]], ssem[k], add=True)

    def scat_wait(k):
        pltpu.make_async_copy(ones_v, dacc.at[didx[k]], ssem[k]).wait()

    for k in range(NSLOT):
        pltpu.sync_copy(dst_hbm.at[w, k], didx[k])

    def body(i, carry):
        for k in range(NSLOT):
            j = i * NSLOT + k

            @pl.when(j < NCH)
            def _():
                scat_start(k)
                jn = j + NSLOT

                @pl.when(jn < NCH)
                def _():
                    scat_wait(k)
                    pltpu.sync_copy(dst_hbm.at[w, jn], didx[k])
        return carry

    lax.fori_loop(0, NITER, body, 0)
    for k in range(NSLOT):
        scat_wait(k)
    plsc.subcore_barrier()
    pltpu.sync_copy(dacc.at[pl.ds(sid * ZROWS, ZROWS)],
                    deg_out.at[cid, pl.ds(sid * ZROWS, ZROWS)])


@functools.cache
def _build_sc():
    mesh = plsc.VectorSubcoreMesh(**_MESH)
    sc_prop = pl.kernel(
        _sc_prop_body,
        out_type=jax.ShapeDtypeStruct((NC, ACC_ROWS, D), jnp.float32),
        mesh=mesh,
        scratch_types=(
            [pltpu.VMEM_SHARED((ACC_ROWS, D), jnp.float32)]
            + [pltpu.VMEM((CHUNK,), jnp.int32)] * (2 * NSLOT)
            + [pltpu.VMEM((CHUNK, D), jnp.float32)] * NSLOT
            + [pltpu.SemaphoreType.DMA] * (2 * NSLOT)
        ),
        name="sc_edge_sum",
    )
    sc_deg = pl.kernel(
        _sc_deg_body,
        out_type=jax.ShapeDtypeStruct((NC, ACC_ROWS, D), jnp.float32),
        mesh=plsc.VectorSubcoreMesh(**_MESH),
        scratch_types=(
            [pltpu.VMEM_SHARED((ACC_ROWS, D), jnp.float32)]
            + [pltpu.VMEM((CHUNK,), jnp.int32)] * NSLOT
            + [pltpu.SemaphoreType.DMA] * NSLOT
            + [pltpu.VMEM((CHUNK, D), jnp.float32)]
        ),
        name="sc_degree",
    )
    return sc_prop, sc_deg


def _pre_body(deg_ref, x_ref, dinv_ref, dinv2_ref, u_ref):
    d = deg_ref[0, :, 0:1] + deg_ref[1, :, 0:1] + 1.0
    di = lax.rsqrt(d)
    dinv_ref[...] = di
    dinv2_ref[...] = di * di
    u_ref[...] = x_ref[...] * di


def _mid_body(acc_ref, u_ref, s_ref, out_ref):
    out_ref[...] = (acc_ref[0] + acc_ref[1] + u_ref[...]) * s_ref[...]


_CBN = (1.0 + EPS) ** -0.5
_DNUMS = (((1,), (1,)), ((), ()))


def _layer_body(acc_ref, u_ref, dinv_ref, w_ref, b_ref, g_ref, be_ref,
                un_ref):
    y = (acc_ref[0] + acc_ref[1] + u_ref[...]) * dinv_ref[...]
    z = lax.dot_general(y, w_ref[...], _DNUMS,
                        preferred_element_type=jnp.float32) + b_ref[...]
    z = z * (_CBN * g_ref[...]) + be_ref[...]
    h = jnp.where(z > 0, z, 0.01 * z)
    un_ref[...] = h * dinv_ref[...]


def _last_body(acc_ref, u_ref, dinv_ref, w_ref, b_ref, g_ref, be_ref,
               x_ref, wsc_ref, bsc_ref, batch_ref, wlin_ref, blin_ref,
               out_ref, pooled):
    i = pl.program_id(0)
    y = (acc_ref[0] + acc_ref[1] + u_ref[...]) * dinv_ref[...]
    z = lax.dot_general(y, w_ref[...], _DNUMS,
                        preferred_element_type=jnp.float32) + b_ref[...]
    z = z * (_CBN * g_ref[...]) + be_ref[...]
    h3 = jnp.where(z > 0, z, 0.01 * z)
    h = (h3 + lax.dot_general(x_ref[...], wsc_ref[...], _DNUMS,
                              preferred_element_type=jnp.float32)
         + bsc_ref[...])

    @pl.when(i == 0)
    def _():
        pooled[...] = jnp.full((G, D), -jnp.inf, jnp.float32)

    b = batch_ref[...]
    for g in range(G):
        hm = jnp.where(b == g, h, -jnp.inf)
        mg = jnp.max(hm, axis=0, keepdims=True)
        pooled[g:g + 1, :] = jnp.maximum(pooled[g:g + 1, :], mg)

    @pl.when(i == GRID - 1)
    def _():
        out_ref[...] = (jnp.dot(pooled[...], wlin_ref[...],
                                preferred_element_type=jnp.float32)
                        + blin_ref[...])


def _spec_rows(width=D):
    return pl.BlockSpec((R, width), lambda i: (i, 0))


def _spec_const(shape):
    return pl.BlockSpec(shape, lambda i: (0, 0))


_SPEC_ACC = pl.BlockSpec((2, R, D), lambda i: (0, i, 0))
_F32 = jnp.float32


@functools.cache
def _build_tc():
    tc_pre = pl.pallas_call(
        _pre_body,
        grid=(GRID,),
        in_specs=[_SPEC_ACC, _spec_rows()],
        out_specs=[_spec_rows(1), _spec_rows(1), _spec_rows()],
        out_shape=[jax.ShapeDtypeStruct((N, 1), _F32),
                   jax.ShapeDtypeStruct((N, 1), _F32),
                   jax.ShapeDtypeStruct((N, D), _F32)],
    )
    tc_mid = pl.pallas_call(
        _mid_body,
        grid=(GRID,),
        in_specs=[_SPEC_ACC, _spec_rows(), _spec_rows(1)],
        out_specs=_spec_rows(),
        out_shape=jax.ShapeDtypeStruct((N, D), _F32),
    )
    tc_layer = pl.pallas_call(
        _layer_body,
        grid=(GRID,),
        in_specs=[_SPEC_ACC, _spec_rows(), _spec_rows(1),
                  _spec_const((D, D)), _spec_const((1, D)),
                  _spec_const((1, D)), _spec_const((1, D))],
        out_specs=_spec_rows(),
        out_shape=jax.ShapeDtypeStruct((N, D), _F32),
    )
    tc_last = pl.pallas_call(
        _last_body,
        grid=(GRID,),
        in_specs=[_SPEC_ACC, _spec_rows(), _spec_rows(1),
                  _spec_const((D, D)), _spec_const((1, D)),
                  _spec_const((1, D)), _spec_const((1, D)),
                  _spec_rows(), _spec_const((D, D)), _spec_const((1, D)),
                  _spec_rows(1), _spec_const((D, 1)), _spec_const((1, 1))],
        out_specs=_spec_const((G, 1)),
        out_shape=jax.ShapeDtypeStruct((G, 1), _F32),
        scratch_shapes=[pltpu.VMEM((G, D), _F32)],
    )
    return tc_pre, tc_mid, tc_layer, tc_last


def kernel(x, edge_index, edge_attr, batch, W1, b1, W2, b2, W3, b3,
           g1, be1, g2, be2, g3, be3, Wsc, bsc, Wlin, blin):
    sc_prop, sc_deg = _build_sc()
    tc_pre, tc_mid, tc_layer, tc_last = _build_tc()

    ei = edge_attr.reshape(2, -1).astype(jnp.int32)
    src3 = ei[0].reshape(NW, NCH, CHUNK)
    dst3 = ei[1].reshape(NW, NCH, CHUNK)

    row = lambda v: v[None, :]
    batch2 = batch.reshape(N, 1).astype(jnp.int32)

    zeros128 = jnp.zeros((ZROWS, D), _F32)
    ones128 = jnp.ones((CHUNK, D), _F32)

    deg_parts = sc_deg(dst3, ones128, zeros128)
    dinv, dinv2, u = tc_pre(deg_parts, x)

    for W, b, g, be in ((W1, b1, g1, be1), (W2, b2, g2, be2)):
        acc = sc_prop(u, src3, dst3, zeros128)
        u = tc_mid(acc, u, dinv2)
        acc = sc_prop(u, src3, dst3, zeros128)
        u = tc_layer(acc, u, dinv, W, row(b), row(g), row(be))

    acc = sc_prop(u, src3, dst3, zeros128)
    u = tc_mid(acc, u, dinv2)
    acc = sc_prop(u, src3, dst3, zeros128)
    return tc_last(acc, u, dinv, W3, row(b3), row(g3), row(be3),
                   x, Wsc, row(bsc), batch2, Wlin.T, row(blin))


# speedup vs baseline: 1.0096x; 1.0096x over previous
import functools

import jax
import jax.numpy as jnp
from jax import lax
from jax.experimental import pallas as pl
from jax.experimental.pallas import tpu as pltpu
from jax.experimental.pallas import tpu_sc as plsc

N = 10000
D = 128
E = 320000
G = 16
EPS = 1e-5

NC = 2
NS = 16
NW = NC * NS
CHUNK = 80
NCH = 125
PER_W = NCH * CHUNK
ACC_ROWS = 10112
ZROWS = ACC_ROWS // NS

R = 2000
GRID = N // R

_MESH = dict(core_axis_name="c", subcore_axis_name="s", num_cores=NC,
             num_subcores=NS)


NSLOT = 4
NITER = (NCH + NSLOT - 1) // NSLOT


def _sc_prop_body(u_hbm, src_hbm, dst_hbm, z_hbm, acc_out, acc_sh, *bufs):
    sidx = bufs[0:NSLOT]
    didx = bufs[NSLOT:2 * NSLOT]
    rows = bufs[2 * NSLOT:3 * NSLOT]
    gsem = bufs[3 * NSLOT:4 * NSLOT]
    ssem = bufs[4 * NSLOT:5 * NSLOT]
    cid = lax.axis_index("c")
    sid = lax.axis_index("s")
    pltpu.sync_copy(z_hbm, acc_sh.at[pl.ds(sid * ZROWS, ZROWS)])
    w = cid * NS + sid
    plsc.subcore_barrier()

    def idx_load(j, k):
        pltpu.sync_copy(src_hbm.at[w, j], sidx[k])
        pltpu.sync_copy(dst_hbm.at[w, j], didx[k])

    def gather_start(k):
        pltpu.async_copy(u_hbm.at[sidx[k]], rows[k], gsem[k])

    def gather_wait(k):
        pltpu.make_async_copy(u_hbm.at[sidx[k]], rows[k], gsem[k]).wait()

    def scat_start(k):
        pltpu.async_copy(rows[k], acc_sh.at[didx[k]], ssem[k], add=True)

    def scat_wait(k):
        pltpu.make_async_copy(rows[k], acc_sh.at[didx[k]], ssem[k]).wait()

    for k in range(NSLOT):
        idx_load(k, k)
        gather_start(k)

    def body(i, carry):
        for k in range(NSLOT):
            j = i * NSLOT + k

            @pl.when(j < NCH)
            def _():
                gather_wait(k)
                scat_start(k)
                jn = j + NSLOT

                @pl.when(jn < NCH)
                def _():
                    scat_wait(k)
                    idx_load(jn, k)
                    gather_start(k)
        return carry

    lax.fori_loop(0, NITER, body, 0)
    for k in range(NSLOT):
        scat_wait(k)
    plsc.subcore_barrier()
    pltpu.sync_copy(acc_sh.at[pl.ds(sid * ZROWS, ZROWS)],
                    acc_out.at[cid, pl.ds(sid * ZROWS, ZROWS)])


def _sc_deg_body(dst_hbm, ones_hbm, z_hbm, deg_out, dacc, *bufs):
    didx = bufs[0:NSLOT]
    ssem = bufs[NSLOT:2 * NSLOT]
    ones_v = bufs[2 * NSLOT]
    cid = lax.axis_index("c")
    sid = lax.axis_index("s")
    pltpu.sync_copy(z_hbm, dacc.at[pl.ds(sid * ZROWS, ZROWS)])
    w = cid * NS + sid
    pltpu.sync_copy(ones_hbm, ones_v)
    plsc.subcore_barrier()

    def scat_start(k):
        pltpu.async_copy(ones_v, dacc.at[didx[k]], ssem[k], add=True)

    def scat_wait(k):
        pltpu.make_async_copy(ones_v, dacc.at[didx[k]], ssem[k]).wait()

    for k in range(NSLOT):
        pltpu.sync_copy(dst_hbm.at[w, k], didx[k])

    def body(i, carry):
        for k in range(NSLOT):
            j = i * NSLOT + k

            @pl.when(j < NCH)
            def _():
                scat_start(k)
                jn = j + NSLOT

                @pl.when(jn < NCH)
                def _():
                    scat_wait(k)
                    pltpu.sync_copy(dst_hbm.at[w, jn], didx[k])
        return carry

    lax.fori_loop(0, NITER, body, 0)
    for k in range(NSLOT):
        scat_wait(k)
    plsc.subcore_barrier()
    pltpu.sync_copy(dacc.at[pl.ds(sid * ZROWS, ZROWS)],
                    deg_out.at[cid, pl.ds(sid * ZROWS, ZROWS)])


@functools.cache
def _build_sc():
    mesh = plsc.VectorSubcoreMesh(**_MESH)
    sc_prop = pl.kernel(
        _sc_prop_body,
        out_type=jax.ShapeDtypeStruct((NC, ACC_ROWS, D), jnp.float32),
        mesh=mesh,
        scratch_types=(
            [pltpu.VMEM_SHARED((ACC_ROWS, D), jnp.float32)]
            + [pltpu.VMEM((CHUNK,), jnp.int32)] * (2 * NSLOT)
            + [pltpu.VMEM((CHUNK, D), jnp.float32)] * NSLOT
            + [pltpu.SemaphoreType.DMA] * (2 * NSLOT)
        ),
        name="sc_edge_sum",
    )
    sc_deg = pl.kernel(
        _sc_deg_body,
        out_type=jax.ShapeDtypeStruct((NC, ACC_ROWS, D), jnp.float32),
        mesh=plsc.VectorSubcoreMesh(**_MESH),
        scratch_types=(
            [pltpu.VMEM_SHARED((ACC_ROWS, D), jnp.float32)]
            + [pltpu.VMEM((CHUNK,), jnp.int32)] * NSLOT
            + [pltpu.SemaphoreType.DMA] * NSLOT
            + [pltpu.VMEM((CHUNK, D), jnp.float32)]
        ),
        name="sc_degree",
    )
    return sc_prop, sc_deg


def _pre_body(deg_ref, x_ref, dinv_ref, dinv2_ref, u_ref):
    d = deg_ref[0, :, 0:1] + deg_ref[1, :, 0:1] + 1.0
    di = lax.rsqrt(d)
    dinv_ref[...] = di
    dinv2_ref[...] = di * di
    u_ref[...] = x_ref[...] * di


def _mid_body(acc_ref, u_ref, s_ref, out_ref):
    out_ref[...] = (acc_ref[0] + acc_ref[1] + u_ref[...]) * s_ref[...]


_CBN = (1.0 + EPS) ** -0.5
_DNUMS = (((1,), (1,)), ((), ()))


def _layer_body(acc_ref, u_ref, dinv_ref, w_ref, b_ref, g_ref, be_ref,
                un_ref):
    y = (acc_ref[0] + acc_ref[1] + u_ref[...]) * dinv_ref[...]
    z = lax.dot_general(y, w_ref[...], _DNUMS,
                        preferred_element_type=jnp.float32) + b_ref[...]
    z = z * (_CBN * g_ref[...]) + be_ref[...]
    h = jnp.where(z > 0, z, 0.01 * z)
    un_ref[...] = h * dinv_ref[...]


def _last_body(acc_ref, u_ref, dinv_ref, w_ref, b_ref, g_ref, be_ref,
               x_ref, wsc_ref, bsc_ref, batch_ref, wlin_ref, blin_ref,
               out_ref, pooled):
    i = pl.program_id(0)
    y = (acc_ref[0] + acc_ref[1] + u_ref[...]) * dinv_ref[...]
    z = lax.dot_general(y, w_ref[...], _DNUMS,
                        preferred_element_type=jnp.float32) + b_ref[...]
    z = z * (_CBN * g_ref[...]) + be_ref[...]
    h3 = jnp.where(z > 0, z, 0.01 * z)
    h = (h3 + lax.dot_general(x_ref[...], wsc_ref[...], _DNUMS,
                              preferred_element_type=jnp.float32)
         + bsc_ref[...])

    @pl.when(i == 0)
    def _():
        pooled[...] = jnp.full((G, D), -jnp.inf, jnp.float32)

    b = batch_ref[...]
    for g in range(G):
        hm = jnp.where(b == g, h, -jnp.inf)
        mg = jnp.max(hm, axis=0, keepdims=True)
        pooled[g:g + 1, :] = jnp.maximum(pooled[g:g + 1, :], mg)

    @pl.when(i == GRID - 1)
    def _():
        out_ref[...] = (jnp.dot(pooled[...], wlin_ref[...],
                                preferred_element_type=jnp.float32)
                        + blin_ref[...])


def _spec_rows(width=D):
    return pl.BlockSpec((R, width), lambda i: (i, 0))


def _spec_const(shape):
    return pl.BlockSpec(shape, lambda i: (0, 0))


_SPEC_ACC = pl.BlockSpec((2, R, D), lambda i: (0, i, 0))
_F32 = jnp.float32


@functools.cache
def _build_tc():
    tc_pre = pl.pallas_call(
        _pre_body,
        grid=(GRID,),
        in_specs=[_SPEC_ACC, _spec_rows()],
        out_specs=[_spec_rows(1), _spec_rows(1), _spec_rows()],
        out_shape=[jax.ShapeDtypeStruct((N, 1), _F32),
                   jax.ShapeDtypeStruct((N, 1), _F32),
                   jax.ShapeDtypeStruct((N, D), _F32)],
    )
    tc_mid = pl.pallas_call(
        _mid_body,
        grid=(GRID,),
        in_specs=[_SPEC_ACC, _spec_rows(), _spec_rows(1)],
        out_specs=_spec_rows(),
        out_shape=jax.ShapeDtypeStruct((N, D), _F32),
    )
    tc_layer = pl.pallas_call(
        _layer_body,
        grid=(GRID,),
        in_specs=[_SPEC_ACC, _spec_rows(), _spec_rows(1),
                  _spec_const((D, D)), _spec_const((1, D)),
                  _spec_const((1, D)), _spec_const((1, D))],
        out_specs=_spec_rows(),
        out_shape=jax.ShapeDtypeStruct((N, D), _F32),
    )
    tc_last = pl.pallas_call(
        _last_body,
        grid=(GRID,),
        in_specs=[_SPEC_ACC, _spec_rows(), _spec_rows(1),
                  _spec_const((D, D)), _spec_const((1, D)),
                  _spec_const((1, D)), _spec_const((1, D)),
                  _spec_rows(), _spec_const((D, D)), _spec_const((1, D)),
                  _spec_rows(1), _spec_const((D, 1)), _spec_const((1, 1))],
        out_specs=_spec_const((G, 1)),
        out_shape=jax.ShapeDtypeStruct((G, 1), _F32),
        scratch_shapes=[pltpu.VMEM((G, D), _F32)],
    )
    return tc_pre, tc_mid, tc_layer, tc_last


def kernel(x, edge_index, edge_attr, batch, W1, b1, W2, b2, W3, b3,
           g1, be1, g2, be2, g3, be3, Wsc, bsc, Wlin, blin):
    sc_prop, sc_deg = _build_sc()
    tc_pre, tc_mid, tc_layer, tc_last = _build_tc()

    ei = edge_attr.reshape(2, -1).astype(jnp.int32)
    src3 = ei[0].reshape(NW, NCH, CHUNK)
    dst3 = ei[1].reshape(NW, NCH, CHUNK)

    row = lambda v: v[None, :]
    batch2 = batch.reshape(N, 1).astype(jnp.int32)

    zeros128 = jnp.zeros((ZROWS, D), _F32)
    ones128 = jnp.ones((CHUNK, D), _F32)

    deg_parts = sc_deg(dst3, ones128, zeros128)
    dinv, dinv2, u = tc_pre(deg_parts, x)

    for W, b, g, be in ((W1, b1, g1, be1), (W2, b2, g2, be2)):
        acc = sc_prop(u, src3, dst3, zeros128)
        u = tc_mid(acc, u, dinv2)
        acc = sc_prop(u, src3, dst3, zeros128)
        u = tc_layer(acc, u, dinv, W, row(b), row(g), row(be))

    acc = sc_prop(u, src3, dst3, zeros128)
    u = tc_mid(acc, u, dinv2)
    acc = sc_prop(u, src3, dst3, zeros128)
    return tc_last(acc, u, dinv, W3, row(b3), row(g3), row(be3),
                   x, Wsc, row(bsc), batch2, Wlin.T, row(blin))
